# baseline (device time: 23427 ns/iter reference)
import jax
import jax.numpy as jnp
from jax import lax
from jax.experimental import pallas as pl
from jax.experimental.pallas import tpu as pltpu

N_DEV = 4
N_EXPERTS = 8


def kernel(x, router_W, route_idx, expert_W, shared_W):
    n_tok, d = x.shape
    n_loc = expert_W.shape[0]
    h = shared_W.shape[1]
    blk = n_tok // N_DEV

    def body(x_ref, rw_ref, idx_ref, ew_ref, sw_ref, out_ref,
             partial_ref, comm_ref, send_sems, recv_sems):
        my = lax.axis_index("i")
        left = (my - 1) % N_DEV
        right = (my + 1) % N_DEV

        barrier_sem = pltpu.get_barrier_semaphore()
        for nbr in (left, right):
            pl.semaphore_signal(
                barrier_sem, inc=1,
                device_id=(nbr,), device_id_type=pl.DeviceIdType.MESH,
            )
        pl.semaphore_wait(barrier_sem, 2)

        xv = x_ref[:, :]
        scores = jnp.dot(xv, rw_ref[:, :], preferred_element_type=jnp.float32)
        s_max = jnp.max(scores, axis=-1, keepdims=True)
        e_s = jnp.exp(scores - s_max)
        probs = e_s / jnp.sum(e_s, axis=-1, keepdims=True)
        idx = idx_ref[:, :]
        eids = lax.broadcasted_iota(jnp.int32, (n_tok, N_EXPERTS), 1)
        p_sel = jnp.sum(jnp.where(idx == eids, probs, 0.0),
                        axis=-1, keepdims=True)

        partial = jnp.zeros((n_tok, h), jnp.float32)
        for el in range(n_loc):
            ge = my * n_loc + el
            scale = jnp.where(idx == ge, p_sel, 0.0)
            partial = partial + jnp.dot(
                xv * scale, ew_ref[el], preferred_element_type=jnp.float32)
        partial_ref[:, :] = partial

        comm_ref[0, :, :] = partial_ref[pl.ds(left * blk, blk), :]
        for s in range(N_DEV - 1):
            rdma = pltpu.make_async_remote_copy(
                src_ref=comm_ref.at[s],
                dst_ref=comm_ref.at[s + 1],
                send_sem=send_sems.at[s],
                recv_sem=recv_sems.at[s],
                device_id=(right,),
                device_id_type=pl.DeviceIdType.MESH,
            )
            rdma.start()
            rdma.wait()
            b = (my - s - 2) % N_DEV
            comm_ref[s + 1, :, :] = (
                comm_ref[s + 1, :, :] + partial_ref[pl.ds(b * blk, blk), :])

        x_own = x_ref[pl.ds(my * blk, blk), :]
        out_ref[:, :] = comm_ref[N_DEV - 1, :, :] + jnp.dot(
            x_own, sw_ref[:, :], preferred_element_type=jnp.float32)

    return pl.pallas_call(
        body,
        out_shape=jax.ShapeDtypeStruct((blk, h), jnp.float32),
        in_specs=[pl.BlockSpec(memory_space=pltpu.VMEM)] * 5,
        out_specs=pl.BlockSpec(memory_space=pltpu.VMEM),
        scratch_shapes=[
            pltpu.VMEM((n_tok, h), jnp.float32),
            pltpu.VMEM((N_DEV, blk, h), jnp.float32),
            pltpu.SemaphoreType.DMA((N_DEV - 1,)),
            pltpu.SemaphoreType.DMA((N_DEV - 1,)),
        ],
        compiler_params=pltpu.CompilerParams(collective_id=0),
    )(x, router_W, route_idx, expert_W, shared_W)


# device time: 17462 ns/iter; 1.3416x vs baseline; 1.3416x over previous
import jax
import jax.numpy as jnp
from jax import lax
from jax.experimental import pallas as pl
from jax.experimental.pallas import tpu as pltpu

N_DEV = 4
N_EXPERTS = 8


def kernel(x, router_W, route_idx, expert_W, shared_W):
    n_tok, d = x.shape
    n_loc = expert_W.shape[0]
    h = shared_W.shape[1]
    blk = n_tok // N_DEV

    def body(x_ref, rw_ref, idx_ref, ew_ref, sw_ref, out_ref,
             xs_ref, send_ref, comm_ref, send_sems, recv_sems):
        my = lax.axis_index("i")
        peers = [(my + j) % N_DEV for j in range(1, N_DEV)]

        barrier_sem = pltpu.get_barrier_semaphore()
        for q in peers:
            pl.semaphore_signal(
                barrier_sem, inc=1,
                device_id=(q,), device_id_type=pl.DeviceIdType.MESH,
            )
        pl.semaphore_wait(barrier_sem, N_DEV - 1)

        xv = x_ref[:, :]
        scores = jnp.dot(xv, rw_ref[:, :], preferred_element_type=jnp.float32)
        s_max = jnp.max(scores, axis=-1, keepdims=True)
        e_s = jnp.exp(scores - s_max)
        probs = e_s / jnp.sum(e_s, axis=-1, keepdims=True)
        idx = idx_ref[:, :]
        eids = lax.broadcasted_iota(jnp.int32, (n_tok, N_EXPERTS), 1)
        p_sel = jnp.sum(jnp.where(idx == eids, probs, 0.0),
                        axis=-1, keepdims=True)

        for el in range(n_loc):
            ge = my * n_loc + el
            scale = jnp.where(idx == ge, p_sel, 0.0)
            xs_ref[el, :, :] = xv * scale

        def partial_block(q):
            acc = jnp.zeros((blk, h), jnp.float32)
            for el in range(n_loc):
                acc = acc + jnp.dot(
                    xs_ref[el, pl.ds(q * blk, blk), :], ew_ref[el],
                    preferred_element_type=jnp.float32)
            return acc

        rdmas = []
        for j, q in enumerate(peers):
            send_ref[j, :, :] = partial_block(q)
            rdma = pltpu.make_async_remote_copy(
                src_ref=send_ref.at[j],
                dst_ref=comm_ref.at[my],
                send_sem=send_sems.at[j],
                recv_sem=recv_sems.at[my],
                device_id=(q,),
                device_id_type=pl.DeviceIdType.MESH,
            )
            rdma.start()
            rdmas.append(rdma)

        comm_ref[my, :, :] = partial_block(my)
        x_own = x_ref[pl.ds(my * blk, blk), :]
        shared_own = jnp.dot(x_own, sw_ref[:, :],
                             preferred_element_type=jnp.float32)

        for j, q in enumerate(peers):
            recv = pltpu.make_async_remote_copy(
                src_ref=send_ref.at[j],
                dst_ref=comm_ref.at[q],
                send_sem=send_sems.at[j],
                recv_sem=recv_sems.at[q],
                device_id=(q,),
                device_id_type=pl.DeviceIdType.MESH,
            )
            recv.wait_recv()

        out_ref[:, :] = (shared_own
                         + comm_ref[0, :, :] + comm_ref[1, :, :]
                         + comm_ref[2, :, :] + comm_ref[3, :, :])

        for rdma in rdmas:
            rdma.wait_send()

    return pl.pallas_call(
        body,
        out_shape=jax.ShapeDtypeStruct((blk, h), jnp.float32),
        in_specs=[pl.BlockSpec(memory_space=pltpu.VMEM)] * 5,
        out_specs=pl.BlockSpec(memory_space=pltpu.VMEM),
        scratch_shapes=[
            pltpu.VMEM((n_loc, n_tok, d), jnp.float32),
            pltpu.VMEM((N_DEV - 1, blk, h), jnp.float32),
            pltpu.VMEM((N_DEV, blk, h), jnp.float32),
            pltpu.SemaphoreType.DMA((N_DEV - 1,)),
            pltpu.SemaphoreType.DMA((N_DEV,)),
        ],
        compiler_params=pltpu.CompilerParams(collective_id=0),
    )(x, router_W, route_idx, expert_W, shared_W)


# device time: 13906 ns/iter; 1.6847x vs baseline; 1.2557x over previous
import jax
import jax.numpy as jnp
from jax import lax
from jax.experimental import pallas as pl
from jax.experimental.pallas import tpu as pltpu

N_DEV = 4
N_EXPERTS = 8


def kernel(x, router_W, route_idx, expert_W, shared_W):
    n_tok, d = x.shape
    n_loc = expert_W.shape[0]
    h = shared_W.shape[1]
    blk = n_tok // N_DEV

    def body(x_ref, rw_ref, idx_ref, ew_ref, sw_ref, out_ref,
             xcat_ref, wcat_ref, send_ref, comm_ref, send_sems, recv_sems):
        my = lax.axis_index("i")
        peers = [(my + 2) % N_DEV, (my + 1) % N_DEV, (my + 3) % N_DEV]

        barrier_sem = pltpu.get_barrier_semaphore()
        for q in peers:
            pl.semaphore_signal(
                barrier_sem, inc=1,
                device_id=(q,), device_id_type=pl.DeviceIdType.MESH,
            )

        xv = x_ref[:, :]
        scores = jnp.dot(xv, rw_ref[:, :], preferred_element_type=jnp.float32)
        s_max = jnp.max(scores, axis=-1, keepdims=True)
        e_s = jnp.exp(scores - s_max)
        probs = e_s / jnp.sum(e_s, axis=-1, keepdims=True)
        idx = idx_ref[:, :]
        eids = lax.broadcasted_iota(jnp.int32, (n_tok, N_EXPERTS), 1)
        p_sel = jnp.sum(jnp.where(idx == eids, probs, 0.0),
                        axis=-1, keepdims=True)

        for el in range(n_loc):
            ge = my * n_loc + el
            scale = jnp.where(idx == ge, p_sel, 0.0)
            xcat_ref[:, el * d:(el + 1) * d] = (xv * scale).astype(jnp.bfloat16)
        wcat_ref[:, :] = ew_ref[:, :, :].reshape(n_loc * d, h).astype(jnp.bfloat16)

        def partial_block(q):
            return jnp.dot(xcat_ref[pl.ds(q * blk, blk), :], wcat_ref[:, :],
                           preferred_element_type=jnp.float32)

        pl.semaphore_wait(barrier_sem, N_DEV - 1)
        rdmas = []
        for j, q in enumerate(peers):
            send_ref[j, :, :] = partial_block(q).astype(jnp.bfloat16)
            rdma = pltpu.make_async_remote_copy(
                src_ref=send_ref.at[j],
                dst_ref=comm_ref.at[my],
                send_sem=send_sems.at[j],
                recv_sem=recv_sems.at[my],
                device_id=(q,),
                device_id_type=pl.DeviceIdType.MESH,
            )
            rdma.start()
            rdmas.append(rdma)

        comm_ref[my, :, :] = partial_block(my).astype(jnp.bfloat16)
        x_own = x_ref[pl.ds(my * blk, blk), :].astype(jnp.bfloat16)
        shared_own = jnp.dot(x_own, sw_ref[:, :].astype(jnp.bfloat16),
                             preferred_element_type=jnp.float32)

        for j, q in enumerate(peers):
            recv = pltpu.make_async_remote_copy(
                src_ref=send_ref.at[j],
                dst_ref=comm_ref.at[q],
                send_sem=send_sems.at[j],
                recv_sem=recv_sems.at[q],
                device_id=(q,),
                device_id_type=pl.DeviceIdType.MESH,
            )
            recv.wait_recv()

        acc = (comm_ref[0, :, :].astype(jnp.float32)
               + comm_ref[1, :, :].astype(jnp.float32)
               + comm_ref[2, :, :].astype(jnp.float32)
               + comm_ref[3, :, :].astype(jnp.float32))
        out_ref[:, :] = shared_own + acc

        for rdma in rdmas:
            rdma.wait_send()

    return pl.pallas_call(
        body,
        out_shape=jax.ShapeDtypeStruct((blk, h), jnp.float32),
        in_specs=[pl.BlockSpec(memory_space=pltpu.VMEM)] * 5,
        out_specs=pl.BlockSpec(memory_space=pltpu.VMEM),
        scratch_shapes=[
            pltpu.VMEM((n_tok, n_loc * d), jnp.bfloat16),
            pltpu.VMEM((n_loc * d, h), jnp.bfloat16),
            pltpu.VMEM((N_DEV - 1, blk, h), jnp.bfloat16),
            pltpu.VMEM((N_DEV, blk, h), jnp.bfloat16),
            pltpu.SemaphoreType.DMA((N_DEV - 1,)),
            pltpu.SemaphoreType.DMA((N_DEV,)),
        ],
        compiler_params=pltpu.CompilerParams(collective_id=0),
    )(x, router_W, route_idx, expert_W, shared_W)


# device time: 11386 ns/iter; 2.0575x vs baseline; 1.2213x over previous
import jax
import jax.numpy as jnp
from jax import lax
from jax.experimental import pallas as pl
from jax.experimental.pallas import tpu as pltpu

N_DEV = 4
N_EXPERTS = 8


def kernel(x, router_W, route_idx, expert_W, shared_W):
    n_tok, d = x.shape
    n_loc = expert_W.shape[0]
    h = shared_W.shape[1]
    blk = n_tok // N_DEV

    def body(x_ref, rw_ref, idx_ref, wcat_ref, sw_ref, out_ref,
             xcat_ref, part_ref, comm_ref, send_sems, recv_sems):
        my = lax.axis_index("i")
        peers = [(my + 2) % N_DEV, (my + 1) % N_DEV, (my + 3) % N_DEV]

        barrier_sem = pltpu.get_barrier_semaphore()
        for q in peers:
            pl.semaphore_signal(
                barrier_sem, inc=1,
                device_id=(q,), device_id_type=pl.DeviceIdType.MESH,
            )

        comm_ref[my, :, :] = jnp.zeros((blk, h), jnp.bfloat16)

        xv = x_ref[:, :]
        scores = jnp.dot(xv, rw_ref[:, :], preferred_element_type=jnp.float32)
        s_max = jnp.max(scores, axis=-1, keepdims=True)
        e_s = jnp.exp(scores - s_max)
        probs = e_s / jnp.sum(e_s, axis=-1, keepdims=True)
        idx = idx_ref[:, :]
        eids = lax.broadcasted_iota(jnp.int32, (n_tok, N_EXPERTS), 1)
        p_sel = jnp.sum(jnp.where(idx == eids, probs, 0.0),
                        axis=-1, keepdims=True)

        for el in range(n_loc):
            ge = my * n_loc + el
            scale = jnp.where(idx == ge, p_sel, 0.0)
            xcat_ref[:, el * d:(el + 1) * d] = (xv * scale).astype(jnp.bfloat16)

        part_ref[:, :] = jnp.dot(
            xcat_ref[:, :], wcat_ref[:, :],
            preferred_element_type=jnp.float32).astype(jnp.bfloat16)

        pl.semaphore_wait(barrier_sem, N_DEV - 1)
        rdmas = []
        for j, q in enumerate(peers):
            rdma = pltpu.make_async_remote_copy(
                src_ref=part_ref.at[pl.ds(q * blk, blk), :],
                dst_ref=comm_ref.at[my],
                send_sem=send_sems.at[j],
                recv_sem=recv_sems.at[my],
                device_id=(q,),
                device_id_type=pl.DeviceIdType.MESH,
            )
            rdma.start()
            rdmas.append(rdma)

        x_own = x_ref[pl.ds(my * blk, blk), :]
        shared_own = jnp.dot(x_own, sw_ref[:, :],
                             preferred_element_type=jnp.float32)
        own = part_ref[pl.ds(my * blk, blk), :].astype(jnp.float32)

        for j, q in enumerate(peers):
            recv = pltpu.make_async_remote_copy(
                src_ref=part_ref.at[pl.ds(q * blk, blk), :],
                dst_ref=comm_ref.at[q],
                send_sem=send_sems.at[j],
                recv_sem=recv_sems.at[q],
                device_id=(q,),
                device_id_type=pl.DeviceIdType.MESH,
            )
            recv.wait_recv()

        acc = (comm_ref[0, :, :].astype(jnp.float32)
               + comm_ref[1, :, :].astype(jnp.float32)
               + comm_ref[2, :, :].astype(jnp.float32)
               + comm_ref[3, :, :].astype(jnp.float32))
        out_ref[:, :] = (shared_own + own + acc).astype(jnp.bfloat16)

        for rdma in rdmas:
            rdma.wait_send()

    wcat = expert_W.reshape(n_loc * d, h).astype(jnp.bfloat16)

    return pl.pallas_call(
        body,
        out_shape=jax.ShapeDtypeStruct((blk, h), jnp.bfloat16),
        in_specs=[pl.BlockSpec(memory_space=pltpu.VMEM)] * 5,
        out_specs=pl.BlockSpec(memory_space=pltpu.VMEM),
        scratch_shapes=[
            pltpu.VMEM((n_tok, n_loc * d), jnp.bfloat16),
            pltpu.VMEM((n_tok, h), jnp.bfloat16),
            pltpu.VMEM((N_DEV, blk, h), jnp.bfloat16),
            pltpu.SemaphoreType.DMA((N_DEV - 1,)),
            pltpu.SemaphoreType.DMA((N_DEV,)),
        ],
        compiler_params=pltpu.CompilerParams(collective_id=0),
    )(x, router_W, route_idx, wcat, shared_W)
